# baseline (speedup 1.0000x reference)
import functools

import jax
import jax.numpy as jnp
from jax import lax
from jax.experimental import pallas as pl
from jax.experimental.pallas import tpu as pltpu
from jax.experimental.pallas import tpu_sc as plsc

N = 10000
NP = 10240
E = 320000
D_IN = 128
D_HID = 128
N_CLASSES = 40
D2P = 128

NC = 2
NS = 16
LANES = 16
CHUNK = 128
SCHUNK = 80
EDGES_PER_TILE = NP
EP = NC * NS * EDGES_PER_TILE
NCHUNKS = EDGES_PER_TILE // CHUNK
SNCHUNKS = EDGES_PER_TILE // SCHUNK
ROWS_PER_TILE = NP // NS
ZR = 16


def _sc_mesh():
    return plsc.VectorSubcoreMesh(core_axis_name="c", subcore_axis_name="s")


def _zero_fill(zbuf, rows, cols):
    @pl.loop(0, rows)
    def _(r):
        @pl.loop(0, cols, step=LANES)
        def _(col):
            zbuf[r, pl.ds(col, LANES)] = jnp.zeros((LANES,), jnp.float32)


def _degrees(src, dst):
    @functools.partial(
        pl.kernel,
        out_type=[
            jax.ShapeDtypeStruct((NC, NP), jnp.float32),
            jax.ShapeDtypeStruct((NC, NP), jnp.float32),
        ],
        mesh=_sc_mesh(),
        scratch_types=[
            pltpu.VMEM((EDGES_PER_TILE,), jnp.int32),
            pltpu.VMEM((EDGES_PER_TILE,), jnp.int32),
            pltpu.VMEM((CHUNK,), jnp.int32),
            pltpu.VMEM((CHUNK,), jnp.int32),
            pltpu.VMEM((CHUNK,), jnp.float32),
            pltpu.VMEM((ROWS_PER_TILE,), jnp.float32),
            pltpu.VMEM_SHARED((NP,), jnp.float32),
            pltpu.VMEM_SHARED((NP,), jnp.float32),
        ],
    )
    def k(src_hbm, dst_hbm, dego_hbm, degi_hbm, sia, dia, si_v, di_v,
          ones_v, z_v, dego_sh, degi_sh):
        c = lax.axis_index("c")
        s = lax.axis_index("s")
        base = (c * NS + s) * EDGES_PER_TILE
        pltpu.sync_copy(src_hbm.at[pl.ds(base, EDGES_PER_TILE)], sia)
        pltpu.sync_copy(dst_hbm.at[pl.ds(base, EDGES_PER_TILE)], dia)

        @pl.loop(0, ROWS_PER_TILE, step=LANES)
        def _(i):
            z_v[pl.ds(i, LANES)] = jnp.zeros((LANES,), jnp.float32)

        @pl.loop(0, CHUNK, step=LANES)
        def _(i):
            ones_v[pl.ds(i, LANES)] = jnp.ones((LANES,), jnp.float32)

        r0 = s * ROWS_PER_TILE
        pltpu.sync_copy(z_v, dego_sh.at[pl.ds(r0, ROWS_PER_TILE)])
        pltpu.sync_copy(z_v, degi_sh.at[pl.ds(r0, ROWS_PER_TILE)])
        plsc.subcore_barrier()

        @pl.loop(0, NCHUNKS)
        def _(j):
            e0 = j * CHUNK

            @pl.loop(0, CHUNK, step=LANES)
            def _(i):
                si_v[pl.ds(i, LANES)] = sia[pl.ds(e0 + i, LANES)]
                di_v[pl.ds(i, LANES)] = dia[pl.ds(e0 + i, LANES)]

            pltpu.sync_copy(ones_v, dego_sh.at[si_v], add=True)
            pltpu.sync_copy(ones_v, degi_sh.at[di_v], add=True)

        plsc.subcore_barrier()
        pltpu.sync_copy(dego_sh.at[pl.ds(r0, ROWS_PER_TILE)],
                        dego_hbm.at[c].at[pl.ds(r0, ROWS_PER_TILE)])
        pltpu.sync_copy(degi_sh.at[pl.ds(r0, ROWS_PER_TILE)],
                        degi_hbm.at[c].at[pl.ds(r0, ROWS_PER_TILE)])

    return k(src, dst)


def _spmm(table, src, dst, d):
    @functools.partial(
        pl.kernel,
        out_type=jax.ShapeDtypeStruct((NC, NP, d), jnp.float32),
        mesh=_sc_mesh(),
        scratch_types=[
            pltpu.VMEM((EDGES_PER_TILE,), jnp.int32),
            pltpu.VMEM((EDGES_PER_TILE,), jnp.int32),
            pltpu.VMEM((SCHUNK,), jnp.int32),
            pltpu.VMEM((SCHUNK,), jnp.int32),
            pltpu.VMEM((SCHUNK,), jnp.int32),
            pltpu.VMEM((SCHUNK,), jnp.int32),
            pltpu.VMEM((SCHUNK, d), jnp.float32),
            pltpu.VMEM((SCHUNK, d), jnp.float32),
            pltpu.VMEM((ZR, d), jnp.float32),
            pltpu.VMEM_SHARED((NP, d), jnp.float32),
            pltpu.SemaphoreType.DMA,
            pltpu.SemaphoreType.DMA,
            pltpu.SemaphoreType.DMA,
            pltpu.SemaphoreType.DMA,
        ],
    )
    def k(tab_hbm, src_hbm, dst_hbm, out_hbm, sia, dia, si0_v, si1_v,
          di0_v, di1_v, rows0, rows1, z_v, acc_sh, gs0, gs1, ss0, ss1):
        c = lax.axis_index("c")
        s = lax.axis_index("s")
        base = (c * NS + s) * EDGES_PER_TILE
        pltpu.sync_copy(src_hbm.at[pl.ds(base, EDGES_PER_TILE)], sia)
        pltpu.sync_copy(dst_hbm.at[pl.ds(base, EDGES_PER_TILE)], dia)
        _zero_fill(z_v, ZR, d)

        @pl.loop(0, ROWS_PER_TILE, step=ZR)
        def _(r):
            pltpu.sync_copy(z_v, acc_sh.at[pl.ds(s * ROWS_PER_TILE + r, ZR)])

        plsc.subcore_barrier()

        def gather(idx_v, e0, rows_v, gsem):
            @pl.loop(0, SCHUNK, step=LANES)
            def _(i):
                idx_v[pl.ds(i, LANES)] = sia[pl.ds(e0 + i, LANES)]

            pltpu.async_copy(tab_hbm.at[idx_v], rows_v, gsem)

        def gwait(idx_v, rows_v, gsem):
            pltpu.make_async_copy(tab_hbm.at[idx_v], rows_v, gsem).wait()

        def scatter(di_v, e0, rows_v, ssem):
            @pl.loop(0, SCHUNK, step=LANES)
            def _(i):
                di_v[pl.ds(i, LANES)] = dia[pl.ds(e0 + i, LANES)]

            pltpu.async_copy(rows_v, acc_sh.at[di_v], ssem, add=True)

        def swait(di_v, rows_v, ssem):
            pltpu.make_async_copy(rows_v, acc_sh.at[di_v], ssem).wait()

        gather(si0_v, 0, rows0, gs0)
        gather(si1_v, SCHUNK, rows1, gs1)

        @pl.loop(0, (SNCHUNKS - 2) * SCHUNK, step=2 * SCHUNK)
        def _(e0):
            gwait(si0_v, rows0, gs0)
            scatter(di0_v, e0, rows0, ss0)
            gwait(si1_v, rows1, gs1)
            scatter(di1_v, e0 + SCHUNK, rows1, ss1)
            swait(di0_v, rows0, ss0)
            gather(si0_v, e0 + 2 * SCHUNK, rows0, gs0)
            swait(di1_v, rows1, ss1)
            gather(si1_v, e0 + 3 * SCHUNK, rows1, gs1)

        gwait(si0_v, rows0, gs0)
        scatter(di0_v, (SNCHUNKS - 2) * SCHUNK, rows0, ss0)
        gwait(si1_v, rows1, gs1)
        scatter(di1_v, (SNCHUNKS - 1) * SCHUNK, rows1, ss1)
        swait(di0_v, rows0, ss0)
        swait(di1_v, rows1, ss1)

        plsc.subcore_barrier()
        r0 = s * ROWS_PER_TILE
        pltpu.sync_copy(acc_sh.at[pl.ds(r0, ROWS_PER_TILE)],
                        out_hbm.at[c].at[pl.ds(r0, ROWS_PER_TILE)])

    return k(table, src, dst)


def _norms(dego_ref, degi_ref):
    deg_o = dego_ref[:, 0:1] + dego_ref[:, 1:2]
    deg_i = degi_ref[:, 0:1] + degi_ref[:, 1:2]
    ns = lax.rsqrt(jnp.maximum(deg_o, 1.0))
    nd = lax.rsqrt(jnp.maximum(deg_i, 1.0))
    return ns, nd


def _tc_first(x, w1, dego, degi):
    def body(x_ref, w_ref, dego_ref, degi_ref, o_ref):
        ns, _ = _norms(dego_ref, degi_ref)
        h = x_ref[...] * ns
        o_ref[...] = jnp.dot(h, w_ref[...], precision=lax.Precision.HIGHEST,
                             preferred_element_type=jnp.float32)

    return pl.pallas_call(
        body, out_shape=jax.ShapeDtypeStruct((NP, D_HID), jnp.float32),
    )(x, w1, dego, degi)


def _tc_mid(p1, b1, w2p, dego, degi):
    def body(p_ref, b_ref, w_ref, dego_ref, degi_ref, o_ref):
        ns, nd = _norms(dego_ref, degi_ref)
        h = p_ref[0] + p_ref[1]
        h = jnp.maximum(h * nd + b_ref[...], 0.0)
        o_ref[...] = jnp.dot(h * ns, w_ref[...],
                             precision=lax.Precision.HIGHEST,
                             preferred_element_type=jnp.float32)

    return pl.pallas_call(
        body, out_shape=jax.ShapeDtypeStruct((NP, D2P), jnp.float32),
    )(p1, b1, w2p, dego, degi)


def _tc_last(p2, b2p, dego, degi):
    def body(p_ref, b_ref, dego_ref, degi_ref, o_ref):
        _, nd = _norms(dego_ref, degi_ref)
        o_ref[...] = (p_ref[0] + p_ref[1]) * nd + b_ref[...]

    return pl.pallas_call(
        body, out_shape=jax.ShapeDtypeStruct((NP, D2P), jnp.float32),
    )(p2, b2p, dego, degi)


def kernel(in_feat, edge_index, W1, b1, W2, b2):
    src = edge_index[0].astype(jnp.int32)
    dst = edge_index[1].astype(jnp.int32)
    pad = N + jnp.arange(EP - E, dtype=jnp.int32) % (NP - N)
    src = jnp.concatenate([src, pad])
    dst = jnp.concatenate([dst, pad])
    xp = jnp.pad(in_feat, ((0, NP - N), (0, 0)))
    w2p = jnp.pad(W2, ((0, 0), (0, D2P - N_CLASSES)))
    b1r = b1.reshape(1, D_HID)
    b2p = jnp.pad(b2, (0, D2P - N_CLASSES)).reshape(1, D2P)

    dego, degi = _degrees(src, dst)
    dego, degi = dego.T, degi.T
    z1 = _tc_first(xp, W1, dego, degi)
    p1 = _spmm(z1, src, dst, D_HID)
    z2 = _tc_mid(p1, b1r, w2p, dego, degi)
    p2 = _spmm(z2, src, dst, D2P)
    out_p = _tc_last(p2, b2p, dego, degi)
    return out_p[:N, :N_CLASSES]


# speedup vs baseline: 1.2058x; 1.2058x over previous
import functools

import jax
import jax.numpy as jnp
from jax import lax
from jax.experimental import pallas as pl
from jax.experimental.pallas import tpu as pltpu
from jax.experimental.pallas import tpu_sc as plsc

N = 10000
NP = 10240
E = 320000
D_IN = 128
D_HID = 128
N_CLASSES = 40
D2P = 128

NC = 2
NS = 16
LANES = 16
CHUNK = 128
SCHUNK = 80
EDGES_PER_TILE = NP
EP = NC * NS * EDGES_PER_TILE
NCHUNKS = EDGES_PER_TILE // CHUNK
SNCHUNKS = EDGES_PER_TILE // SCHUNK
ROWS_PER_TILE = NP // NS
ZR = 16


def _sc_mesh():
    return plsc.VectorSubcoreMesh(core_axis_name="c", subcore_axis_name="s")


def _zero_fill(zbuf, rows, cols):
    @pl.loop(0, rows)
    def _(r):
        @pl.loop(0, cols, step=LANES)
        def _(col):
            zbuf[r, pl.ds(col, LANES)] = jnp.zeros((LANES,), jnp.float32)


def _degrees(src, dst):
    @functools.partial(
        pl.kernel,
        out_type=[
            jax.ShapeDtypeStruct((NC, NP), jnp.float32),
            jax.ShapeDtypeStruct((NC, NP), jnp.float32),
        ],
        mesh=_sc_mesh(),
        scratch_types=[
            pltpu.VMEM((EDGES_PER_TILE,), jnp.int32),
            pltpu.VMEM((EDGES_PER_TILE,), jnp.int32),
            pltpu.VMEM((CHUNK,), jnp.int32),
            pltpu.VMEM((CHUNK,), jnp.int32),
            pltpu.VMEM((CHUNK,), jnp.float32),
            pltpu.VMEM((ROWS_PER_TILE,), jnp.float32),
            pltpu.VMEM_SHARED((NP,), jnp.float32),
            pltpu.VMEM_SHARED((NP,), jnp.float32),
        ],
    )
    def k(src_hbm, dst_hbm, dego_hbm, degi_hbm, sia, dia, si_v, di_v,
          ones_v, z_v, dego_sh, degi_sh):
        c = lax.axis_index("c")
        s = lax.axis_index("s")
        base = (c * NS + s) * EDGES_PER_TILE
        pltpu.sync_copy(src_hbm.at[pl.ds(base, EDGES_PER_TILE)], sia)
        pltpu.sync_copy(dst_hbm.at[pl.ds(base, EDGES_PER_TILE)], dia)

        @pl.loop(0, ROWS_PER_TILE, step=LANES)
        def _(i):
            z_v[pl.ds(i, LANES)] = jnp.zeros((LANES,), jnp.float32)

        @pl.loop(0, CHUNK, step=LANES)
        def _(i):
            ones_v[pl.ds(i, LANES)] = jnp.ones((LANES,), jnp.float32)

        r0 = s * ROWS_PER_TILE
        pltpu.sync_copy(z_v, dego_sh.at[pl.ds(r0, ROWS_PER_TILE)])
        pltpu.sync_copy(z_v, degi_sh.at[pl.ds(r0, ROWS_PER_TILE)])
        plsc.subcore_barrier()

        @pl.loop(0, NCHUNKS)
        def _(j):
            e0 = j * CHUNK

            @pl.loop(0, CHUNK, step=LANES)
            def _(i):
                si_v[pl.ds(i, LANES)] = sia[pl.ds(e0 + i, LANES)]
                di_v[pl.ds(i, LANES)] = dia[pl.ds(e0 + i, LANES)]

            pltpu.sync_copy(ones_v, dego_sh.at[si_v], add=True)
            pltpu.sync_copy(ones_v, degi_sh.at[di_v], add=True)

        plsc.subcore_barrier()
        pltpu.sync_copy(dego_sh.at[pl.ds(r0, ROWS_PER_TILE)],
                        dego_hbm.at[c].at[pl.ds(r0, ROWS_PER_TILE)])
        pltpu.sync_copy(degi_sh.at[pl.ds(r0, ROWS_PER_TILE)],
                        degi_hbm.at[c].at[pl.ds(r0, ROWS_PER_TILE)])

    return k(src, dst)


def _spmm(table, src, dst, d):
    @functools.partial(
        pl.kernel,
        out_type=jax.ShapeDtypeStruct((NC, NP, d), jnp.float32),
        mesh=_sc_mesh(),
        scratch_types=[
            pltpu.VMEM((EDGES_PER_TILE,), jnp.int32),
            pltpu.VMEM((EDGES_PER_TILE,), jnp.int32),
            pltpu.VMEM((SCHUNK,), jnp.int32),
            pltpu.VMEM((SCHUNK,), jnp.int32),
            pltpu.VMEM((SCHUNK,), jnp.int32),
            pltpu.VMEM((SCHUNK, d), jnp.float32),
            pltpu.VMEM((SCHUNK, d), jnp.float32),
            pltpu.VMEM((ZR, d), jnp.float32),
            pltpu.VMEM_SHARED((NP, d), jnp.float32),
            pltpu.SemaphoreType.DMA,
            pltpu.SemaphoreType.DMA,
        ],
    )
    def k(tab_hbm, src_hbm, dst_hbm, out_hbm, sia, dia, si0_v, si1_v,
          di_v, rows0, rows1, z_v, acc_sh, sem0, sem1):
        c = lax.axis_index("c")
        s = lax.axis_index("s")
        base = (c * NS + s) * EDGES_PER_TILE
        pltpu.sync_copy(src_hbm.at[pl.ds(base, EDGES_PER_TILE)], sia)
        pltpu.sync_copy(dst_hbm.at[pl.ds(base, EDGES_PER_TILE)], dia)
        _zero_fill(z_v, ZR, d)

        @pl.loop(0, ROWS_PER_TILE, step=ZR)
        def _(r):
            pltpu.sync_copy(z_v, acc_sh.at[pl.ds(s * ROWS_PER_TILE + r, ZR)])

        plsc.subcore_barrier()

        def gather(idx_v, e0, rows_v, sem):
            @pl.loop(0, SCHUNK, step=LANES)
            def _(i):
                idx_v[pl.ds(i, LANES)] = sia[pl.ds(e0 + i, LANES)]

            pltpu.async_copy(tab_hbm.at[idx_v], rows_v, sem)

        def scatter(rows_v, e0):
            @pl.loop(0, SCHUNK, step=LANES)
            def _(i):
                di_v[pl.ds(i, LANES)] = dia[pl.ds(e0 + i, LANES)]

            pltpu.sync_copy(rows_v, acc_sh.at[di_v], add=True)

        def wait(idx_v, rows_v, sem):
            pltpu.make_async_copy(tab_hbm.at[idx_v], rows_v, sem).wait()

        gather(si0_v, 0, rows0, sem0)

        @pl.loop(0, (SNCHUNKS - 2) * SCHUNK, step=2 * SCHUNK)
        def _(e0):
            gather(si1_v, e0 + SCHUNK, rows1, sem1)
            wait(si0_v, rows0, sem0)
            scatter(rows0, e0)
            gather(si0_v, e0 + 2 * SCHUNK, rows0, sem0)
            wait(si1_v, rows1, sem1)
            scatter(rows1, e0 + SCHUNK)

        gather(si1_v, (SNCHUNKS - 1) * SCHUNK, rows1, sem1)
        wait(si0_v, rows0, sem0)
        scatter(rows0, (SNCHUNKS - 2) * SCHUNK)
        wait(si1_v, rows1, sem1)
        scatter(rows1, (SNCHUNKS - 1) * SCHUNK)

        plsc.subcore_barrier()
        r0 = s * ROWS_PER_TILE
        pltpu.sync_copy(acc_sh.at[pl.ds(r0, ROWS_PER_TILE)],
                        out_hbm.at[c].at[pl.ds(r0, ROWS_PER_TILE)])

    return k(table, src, dst)


def _norms(dego_ref, degi_ref):
    deg_o = dego_ref[:, 0:1] + dego_ref[:, 1:2]
    deg_i = degi_ref[:, 0:1] + degi_ref[:, 1:2]
    ns = lax.rsqrt(jnp.maximum(deg_o, 1.0))
    nd = lax.rsqrt(jnp.maximum(deg_i, 1.0))
    return ns, nd


def _tc_first(x, w1, dego, degi):
    def body(x_ref, w_ref, dego_ref, degi_ref, o_ref):
        ns, _ = _norms(dego_ref, degi_ref)
        h = x_ref[...] * ns
        o_ref[...] = jnp.dot(h, w_ref[...], precision=lax.Precision.HIGHEST,
                             preferred_element_type=jnp.float32)

    return pl.pallas_call(
        body, out_shape=jax.ShapeDtypeStruct((NP, D_HID), jnp.float32),
    )(x, w1, dego, degi)


def _tc_mid(p1, b1, w2p, dego, degi):
    def body(p_ref, b_ref, w_ref, dego_ref, degi_ref, o_ref):
        ns, nd = _norms(dego_ref, degi_ref)
        h = p_ref[0] + p_ref[1]
        h = jnp.maximum(h * nd + b_ref[...], 0.0)
        o_ref[...] = jnp.dot(h * ns, w_ref[...],
                             precision=lax.Precision.HIGHEST,
                             preferred_element_type=jnp.float32)

    return pl.pallas_call(
        body, out_shape=jax.ShapeDtypeStruct((NP, D2P), jnp.float32),
    )(p1, b1, w2p, dego, degi)


def _tc_last(p2, b2p, dego, degi):
    def body(p_ref, b_ref, dego_ref, degi_ref, o_ref):
        _, nd = _norms(dego_ref, degi_ref)
        o_ref[...] = (p_ref[0] + p_ref[1]) * nd + b_ref[...]

    return pl.pallas_call(
        body, out_shape=jax.ShapeDtypeStruct((NP, D2P), jnp.float32),
    )(p2, b2p, dego, degi)


def kernel(in_feat, edge_index, W1, b1, W2, b2):
    src = edge_index[0].astype(jnp.int32)
    dst = edge_index[1].astype(jnp.int32)
    pad = N + jnp.arange(EP - E, dtype=jnp.int32) % (NP - N)
    src = jnp.concatenate([src, pad])
    dst = jnp.concatenate([dst, pad])
    xp = jnp.pad(in_feat, ((0, NP - N), (0, 0)))
    w2p = jnp.pad(W2, ((0, 0), (0, D2P - N_CLASSES)))
    b1r = b1.reshape(1, D_HID)
    b2p = jnp.pad(b2, (0, D2P - N_CLASSES)).reshape(1, D2P)

    dego, degi = _degrees(src, dst)
    dego, degi = dego.T, degi.T
    z1 = _tc_first(xp, W1, dego, degi)
    p1 = _spmm(z1, src, dst, D_HID)
    z2 = _tc_mid(p1, b1r, w2p, dego, degi)
    p2 = _spmm(z2, src, dst, D2P)
    out_p = _tc_last(p2, b2p, dego, degi)
    return out_p[:N, :N_CLASSES]
